# baseline (speedup 1.0000x reference)
import jax
import jax.numpy as jnp
from jax.experimental import pallas as pl
from jax.experimental.pallas import tpu as pltpu

_N = 4096
_C = 128
_BLK = 512
_NBLK = _N // _BLK
_B1 = 128
_NB1 = _N // _B1


def _dot_t(a, b):
    return jax.lax.dot_general(
        a, b, (((0,), (0,)), ((), ())),
        preferred_element_type=jnp.float32)


def _split(x):
    hi = x.astype(jnp.bfloat16)
    lo = (x - hi.astype(jnp.float32)).astype(jnp.bfloat16)
    return hi, lo


def _lin2(a, b, Wl, Wr, bias):
    return (a[:, 0:1] * Wl[:, 0][None, :] + a[:, 1:2] * Wl[:, 1][None, :]
            + b[:, 0:1] * Wr[:, 0][None, :] + b[:, 1:2] * Wr[:, 1][None, :]
            + bias)


def _body(a_ref, nc_ref,
          w1l_ref, w1r_ref, b1_ref, w2l_ref, w2r_ref, b2_ref,
          w3l_ref, w3r_ref, b3_ref,
          x3_ref, arg_ref, ll_ref, ent_ref, xout_ref,
          abf_ref, atx_ref, shi_ref, slo_ref, adj_ref, scal_ref, xaug_ref):
    k = pl.program_id(0)

    @pl.when(k < _NB1)
    def _pass1():
        base = k * _B1
        ab = a_ref[...].astype(jnp.bfloat16)
        abf_ref[pl.ds(base, _B1), :] = ab
        nc = nc_ref[pl.ds(base, _B1), :]
        part = _dot_t(ab, nc)
        @pl.when(k == 0)
        def _():
            atx_ref[...] = part
        @pl.when(k > 0)
        def _():
            atx_ref[...] += part

    @pl.when(k == _NB1 - 1)
    def _mid():
        atx6 = atx_ref[...]
        atx = atx6[:, 0:3] + atx6[:, 3:6]
        nodes = (nc_ref[:, 0:2].astype(jnp.float32)
                 + nc_ref[:, 3:5].astype(jnp.float32))
        colsum = atx[:, 2:3]
        deg = jnp.maximum(colsum, 1.0)
        agg = atx[:, 0:2] / deg

        x1 = _lin2(agg, nodes, w1l_ref[...], w1r_ref[...], b1_ref[...])
        logits = _lin2(agg, nodes, w2l_ref[...], w2r_ref[...], b2_ref[...])

        m = jnp.max(logits, axis=-1, keepdims=True)
        e = jnp.exp(logits - m)
        z = jnp.sum(e, axis=-1, keepdims=True)
        s = e / z
        shi, slo = _split(s)
        shi_ref[...] = shi
        slo_ref[...] = slo

        ent_rows = jnp.log(z) - jnp.sum(e * (logits - m), axis=-1,
                                        keepdims=True) / z
        ent_ref[...] = jnp.reshape(jnp.sum(ent_rows) / _N, (1, 1))

        x1h, x1l = _split(x1)
        xout = _dot_t(shi, x1h) + _dot_t(shi, x1l) + _dot_t(slo, x1h)
        xout_ref[...] = xout
        xaug_ref[...] = jnp.concatenate(
            [xout, jnp.ones((_C, 1), jnp.float32)], axis=1)

        g = _dot_t(shi, shi) + _dot_t(shi, slo) + _dot_t(slo, shi)
        gnorm2 = jnp.sum(g * g)
        suma2 = jnp.sum(colsum)
        scal_ref[...] = jnp.concatenate(
            [jnp.reshape(suma2, (1, 1)), jnp.reshape(gnorm2, (1, 1))], axis=1)

    @pl.when(k >= _NB1)
    def _pass2():
        j = k - _NB1
        ab = abf_ref[pl.ds(j * _BLK, _BLK), :]
        y = (jnp.dot(ab, shi_ref[...], preferred_element_type=jnp.float32)
             + jnp.dot(ab, slo_ref[...], preferred_element_type=jnp.float32))
        yh, yl = _split(y)
        sh = shi_ref[pl.ds(j * _BLK, _BLK), :]
        sl = slo_ref[pl.ds(j * _BLK, _BLK), :]
        part = _dot_t(sh, yh) + _dot_t(sh, yl) + _dot_t(sl, yh)
        @pl.when(j == 0)
        def _():
            adj_ref[...] = part
        @pl.when(j > 0)
        def _():
            adj_ref[...] += part

    @pl.when(k == _NB1 + _NBLK - 1)
    def _final():
        adj_p = adj_ref[...]
        rows = jax.lax.broadcasted_iota(jnp.int32, (_C, _C), 0)
        cols = jax.lax.broadcasted_iota(jnp.int32, (_C, _C), 1)

        tr = jnp.sum(jnp.where(rows == cols, adj_p, 0.0))
        suma2 = scal_ref[0, 0]
        gnorm2 = scal_ref[0, 1]
        resid = jnp.maximum(suma2 - 2.0 * tr + gnorm2, 0.0)
        ll_ref[...] = jnp.reshape(jnp.sqrt(resid) / (_N * _N), (1, 1))

        row_max = jnp.max(adj_p, axis=1, keepdims=True)
        is_max = adj_p == row_max
        hard = is_max.astype(jnp.float32)
        arg_ref[...] = jnp.min(jnp.where(is_max, cols, _C), axis=1,
                               keepdims=True)

        xaug = xaug_ref[...]
        agg_aug = _dot_t(hard, xaug)
        deg3 = jnp.maximum(agg_aug[:, 2:3], 1.0)
        agg3 = agg_aug[:, 0:2] / deg3
        x3_ref[...] = _lin2(agg3, xaug[:, 0:2], w3l_ref[...], w3r_ref[...],
                            b3_ref[...])


def kernel(nodes, adjs, W1_l, W1_r, b1, W2_l, W2_r, b2, W3_l, W3_r, b3):
    naug = jnp.concatenate(
        [nodes, jnp.ones((_N, 1), jnp.float32)], axis=1)
    naug_hi = naug.astype(jnp.bfloat16)
    naug_lo = (naug - naug_hi.astype(jnp.float32)).astype(jnp.bfloat16)
    naug_cat = jnp.concatenate([naug_hi, naug_lo], axis=1)

    x3, arg, ll, ent, xout = pl.pallas_call(
        _body,
        grid=(_NB1 + _NBLK,),
        in_specs=[
            pl.BlockSpec((_B1, _N),
                         lambda k: (jnp.minimum(k, _NB1 - 1), 0)),
            pl.BlockSpec((_N, 6), lambda k: (0, 0)),
            pl.BlockSpec((2, 2), lambda k: (0, 0)),
            pl.BlockSpec((2, 2), lambda k: (0, 0)),
            pl.BlockSpec((1, 2), lambda k: (0, 0)),
            pl.BlockSpec((_C, 2), lambda k: (0, 0)),
            pl.BlockSpec((_C, 2), lambda k: (0, 0)),
            pl.BlockSpec((1, _C), lambda k: (0, 0)),
            pl.BlockSpec((1, 2), lambda k: (0, 0)),
            pl.BlockSpec((1, 2), lambda k: (0, 0)),
            pl.BlockSpec((1, 1), lambda k: (0, 0)),
        ],
        out_specs=[
            pl.BlockSpec((_C, 1), lambda k: (0, 0)),
            pl.BlockSpec((_C, 1), lambda k: (0, 0)),
            pl.BlockSpec((1, 1), lambda k: (0, 0)),
            pl.BlockSpec((1, 1), lambda k: (0, 0)),
            pl.BlockSpec((_C, 2), lambda k: (0, 0)),
        ],
        out_shape=[
            jax.ShapeDtypeStruct((_C, 1), jnp.float32),
            jax.ShapeDtypeStruct((_C, 1), jnp.int32),
            jax.ShapeDtypeStruct((1, 1), jnp.float32),
            jax.ShapeDtypeStruct((1, 1), jnp.float32),
            jax.ShapeDtypeStruct((_C, 2), jnp.float32),
        ],
        scratch_shapes=[
            pltpu.VMEM((_N, _N), jnp.bfloat16),
            pltpu.VMEM((_N, 6), jnp.float32),
            pltpu.VMEM((_N, _C), jnp.bfloat16),
            pltpu.VMEM((_N, _C), jnp.bfloat16),
            pltpu.VMEM((_C, _C), jnp.float32),
            pltpu.VMEM((1, 2), jnp.float32),
            pltpu.VMEM((_C, 3), jnp.float32),
        ],
        compiler_params=pltpu.CompilerParams(
            dimension_semantics=("arbitrary",)),
    )(adjs, naug_cat,
      W1_l, W1_r, b1.reshape(1, 2), W2_l, W2_r, b2.reshape(1, _C),
      W3_l, W3_r, b3.reshape(1, 1))

    x3_out = x3[:, 0]
    edge_index = jnp.stack(
        [jnp.arange(_C, dtype=jnp.int32), arg.reshape(_C)])
    return (x3_out, edge_index, ll.reshape(()), ent.reshape(()), xout)


# speedup vs baseline: 1.1126x; 1.1126x over previous
import jax
import jax.numpy as jnp
from jax.experimental import pallas as pl
from jax.experimental.pallas import tpu as pltpu

_N = 4096
_C = 128
_BLK = 512
_NBLK = _N // _BLK
_B1 = 256
_NB1 = _N // _B1


def _dot_t(a, b):
    return jax.lax.dot_general(
        a, b, (((0,), (0,)), ((), ())),
        preferred_element_type=jnp.float32)


def _split(x):
    hi = x.astype(jnp.bfloat16)
    lo = (x - hi.astype(jnp.float32)).astype(jnp.bfloat16)
    return hi, lo


def _lin2(a, b, Wl, Wr, bias):
    return (a[:, 0:1] * Wl[:, 0][None, :] + a[:, 1:2] * Wl[:, 1][None, :]
            + b[:, 0:1] * Wr[:, 0][None, :] + b[:, 1:2] * Wr[:, 1][None, :]
            + bias)


def _body(a_ref, nhi_ref, nlo_ref,
          w1l_ref, w1r_ref, b1_ref, w2l_ref, w2r_ref, b2_ref,
          w3l_ref, w3r_ref, b3_ref,
          x3_ref, arg_ref, ll_ref, ent_ref, xout_ref,
          abf_ref, atx_ref, shi_ref, slo_ref, adj_ref, scal_ref, xaug_ref):
    k = pl.program_id(0)

    @pl.when(k < _NB1)
    def _pass1():
        base = k * _B1
        ab = a_ref[...].astype(jnp.bfloat16)
        abf_ref[pl.ds(base, _B1), :] = ab
        nh = nhi_ref[pl.ds(base, _B1), :]
        nl = nlo_ref[pl.ds(base, _B1), :]
        part = _dot_t(ab, nh) + _dot_t(ab, nl)
        @pl.when(k == 0)
        def _():
            atx_ref[...] = part
        @pl.when(k > 0)
        def _():
            atx_ref[...] += part

    @pl.when(k == _NB1 - 1)
    def _mid():
        atx = atx_ref[...]
        nodes = (nhi_ref[:, 0:2].astype(jnp.float32)
                 + nlo_ref[:, 0:2].astype(jnp.float32))
        colsum = atx[:, 2:3]
        deg = jnp.maximum(colsum, 1.0)
        agg = atx[:, 0:2] / deg

        x1 = _lin2(agg, nodes, w1l_ref[...], w1r_ref[...], b1_ref[...])
        logits = _lin2(agg, nodes, w2l_ref[...], w2r_ref[...], b2_ref[...])

        m = jnp.max(logits, axis=-1, keepdims=True)
        e = jnp.exp(logits - m)
        z = jnp.sum(e, axis=-1, keepdims=True)
        s = e / z
        shi, slo = _split(s)
        shi_ref[...] = shi
        slo_ref[...] = slo

        ent_rows = jnp.log(z) - jnp.sum(e * (logits - m), axis=-1,
                                        keepdims=True) / z
        ent_ref[...] = jnp.reshape(jnp.sum(ent_rows) / _N, (1, 1))

        x1h, x1l = _split(x1)
        xout = _dot_t(shi, x1h) + _dot_t(shi, x1l) + _dot_t(slo, x1h)
        xout_ref[...] = xout
        xaug_ref[...] = jnp.concatenate(
            [xout, jnp.ones((_C, 1), jnp.float32)], axis=1)

        g = _dot_t(shi, shi) + _dot_t(shi, slo) + _dot_t(slo, shi)
        gnorm2 = jnp.sum(g * g)
        suma2 = jnp.sum(colsum)
        scal_ref[...] = jnp.concatenate(
            [jnp.reshape(suma2, (1, 1)), jnp.reshape(gnorm2, (1, 1))], axis=1)

    @pl.when(k >= _NB1)
    def _pass2():
        j = k - _NB1
        ab = abf_ref[pl.ds(j * _BLK, _BLK), :]
        y = (jnp.dot(ab, shi_ref[...], preferred_element_type=jnp.float32)
             + jnp.dot(ab, slo_ref[...], preferred_element_type=jnp.float32))
        yh, yl = _split(y)
        sh = shi_ref[pl.ds(j * _BLK, _BLK), :]
        sl = slo_ref[pl.ds(j * _BLK, _BLK), :]
        part = _dot_t(sh, yh) + _dot_t(sh, yl) + _dot_t(sl, yh)
        @pl.when(j == 0)
        def _():
            adj_ref[...] = part
        @pl.when(j > 0)
        def _():
            adj_ref[...] += part

    @pl.when(k == _NB1 + _NBLK - 1)
    def _final():
        adj_p = adj_ref[...]
        rows = jax.lax.broadcasted_iota(jnp.int32, (_C, _C), 0)
        cols = jax.lax.broadcasted_iota(jnp.int32, (_C, _C), 1)

        tr = jnp.sum(jnp.where(rows == cols, adj_p, 0.0))
        suma2 = scal_ref[0, 0]
        gnorm2 = scal_ref[0, 1]
        resid = jnp.maximum(suma2 - 2.0 * tr + gnorm2, 0.0)
        ll_ref[...] = jnp.reshape(jnp.sqrt(resid) / (_N * _N), (1, 1))

        row_max = jnp.max(adj_p, axis=1, keepdims=True)
        is_max = adj_p == row_max
        hard = is_max.astype(jnp.float32)
        arg_ref[...] = jnp.min(jnp.where(is_max, cols, _C), axis=1,
                               keepdims=True)

        xaug = xaug_ref[...]
        agg_aug = _dot_t(hard, xaug)
        deg3 = jnp.maximum(agg_aug[:, 2:3], 1.0)
        agg3 = agg_aug[:, 0:2] / deg3
        x3_ref[...] = _lin2(agg3, xaug[:, 0:2], w3l_ref[...], w3r_ref[...],
                            b3_ref[...])


def kernel(nodes, adjs, W1_l, W1_r, b1, W2_l, W2_r, b2, W3_l, W3_r, b3):
    naug = jnp.concatenate(
        [nodes, jnp.ones((_N, 1), jnp.float32)], axis=1)
    naug_hi = naug.astype(jnp.bfloat16)
    naug_lo = (naug - naug_hi.astype(jnp.float32)).astype(jnp.bfloat16)

    x3, arg, ll, ent, xout = pl.pallas_call(
        _body,
        grid=(_NB1 + _NBLK,),
        in_specs=[
            pl.BlockSpec((_B1, _N),
                         lambda k: (jnp.minimum(k, _NB1 - 1), 0)),
            pl.BlockSpec((_N, 3), lambda k: (0, 0)),
            pl.BlockSpec((_N, 3), lambda k: (0, 0)),
            pl.BlockSpec((2, 2), lambda k: (0, 0)),
            pl.BlockSpec((2, 2), lambda k: (0, 0)),
            pl.BlockSpec((1, 2), lambda k: (0, 0)),
            pl.BlockSpec((_C, 2), lambda k: (0, 0)),
            pl.BlockSpec((_C, 2), lambda k: (0, 0)),
            pl.BlockSpec((1, _C), lambda k: (0, 0)),
            pl.BlockSpec((1, 2), lambda k: (0, 0)),
            pl.BlockSpec((1, 2), lambda k: (0, 0)),
            pl.BlockSpec((1, 1), lambda k: (0, 0)),
        ],
        out_specs=[
            pl.BlockSpec((_C, 1), lambda k: (0, 0)),
            pl.BlockSpec((_C, 1), lambda k: (0, 0)),
            pl.BlockSpec((1, 1), lambda k: (0, 0)),
            pl.BlockSpec((1, 1), lambda k: (0, 0)),
            pl.BlockSpec((_C, 2), lambda k: (0, 0)),
        ],
        out_shape=[
            jax.ShapeDtypeStruct((_C, 1), jnp.float32),
            jax.ShapeDtypeStruct((_C, 1), jnp.int32),
            jax.ShapeDtypeStruct((1, 1), jnp.float32),
            jax.ShapeDtypeStruct((1, 1), jnp.float32),
            jax.ShapeDtypeStruct((_C, 2), jnp.float32),
        ],
        scratch_shapes=[
            pltpu.VMEM((_N, _N), jnp.bfloat16),
            pltpu.VMEM((_N, 3), jnp.float32),
            pltpu.VMEM((_N, _C), jnp.bfloat16),
            pltpu.VMEM((_N, _C), jnp.bfloat16),
            pltpu.VMEM((_C, _C), jnp.float32),
            pltpu.VMEM((1, 2), jnp.float32),
            pltpu.VMEM((_C, 3), jnp.float32),
        ],
        compiler_params=pltpu.CompilerParams(
            dimension_semantics=("arbitrary",)),
    )(adjs, naug_hi, naug_lo,
      W1_l, W1_r, b1.reshape(1, 2), W2_l, W2_r, b2.reshape(1, _C),
      W3_l, W3_r, b3.reshape(1, 1))

    x3_out = x3[:, 0]
    edge_index = jnp.stack(
        [jnp.arange(_C, dtype=jnp.int32), arg.reshape(_C)])
    return (x3_out, edge_index, ll.reshape(()), ent.reshape(()), xout)


# speedup vs baseline: 1.1790x; 1.0597x over previous
import jax
import jax.numpy as jnp
from jax.experimental import pallas as pl
from jax.experimental.pallas import tpu as pltpu

_N = 4096
_C = 128
_BLK = 512
_NBLK = _N // _BLK


def _dot_t(a, b):
    return jax.lax.dot_general(
        a, b, (((0,), (0,)), ((), ())),
        preferred_element_type=jnp.float32)


def _split(x):
    hi = x.astype(jnp.bfloat16)
    lo = (x - hi.astype(jnp.float32)).astype(jnp.bfloat16)
    return hi, lo


def _lin2(a, b, Wl, Wr, bias):
    return (a[:, 0:1] * Wl[:, 0][None, :] + a[:, 1:2] * Wl[:, 1][None, :]
            + b[:, 0:1] * Wr[:, 0][None, :] + b[:, 1:2] * Wr[:, 1][None, :]
            + bias)


def _body(a_ref, nhi_ref, nlo_ref,
          w1l_ref, w1r_ref, b1_ref, w2l_ref, w2r_ref, b2_ref,
          w3l_ref, w3r_ref, b3_ref,
          x3_ref, arg_ref, ll_ref, ent_ref, xout_ref,
          atx_ref, shi_ref, slo_ref, adj_ref, scal_ref, xaug_ref):
    k = pl.program_id(0)

    @pl.when(k < _NBLK)
    def _pass1():
        base = k * _BLK
        ab = a_ref[...].astype(jnp.bfloat16)
        nh = nhi_ref[pl.ds(base, _BLK), :]
        nl = nlo_ref[pl.ds(base, _BLK), :]
        part = _dot_t(ab, nh) + _dot_t(ab, nl)
        @pl.when(k == 0)
        def _():
            atx_ref[...] = part
        @pl.when(k > 0)
        def _():
            atx_ref[...] += part

    @pl.when(k == _NBLK - 1)
    def _mid():
        atx = atx_ref[...]
        nodes = (nhi_ref[:, 0:2].astype(jnp.float32)
                 + nlo_ref[:, 0:2].astype(jnp.float32))
        colsum = atx[:, 2:3]
        deg = jnp.maximum(colsum, 1.0)
        agg = atx[:, 0:2] / deg

        x1 = _lin2(agg, nodes, w1l_ref[...], w1r_ref[...], b1_ref[...])
        logits = _lin2(agg, nodes, w2l_ref[...], w2r_ref[...], b2_ref[...])

        m = jnp.max(logits, axis=-1, keepdims=True)
        e = jnp.exp(logits - m)
        z = jnp.sum(e, axis=-1, keepdims=True)
        s = e / z
        shi, slo = _split(s)
        shi_ref[...] = shi
        slo_ref[...] = slo

        ent_rows = jnp.log(z) - jnp.sum(e * (logits - m), axis=-1,
                                        keepdims=True) / z
        ent_ref[...] = jnp.reshape(jnp.sum(ent_rows) / _N, (1, 1))

        x1h, x1l = _split(x1)
        xout = _dot_t(shi, x1h) + _dot_t(shi, x1l) + _dot_t(slo, x1h)
        xout_ref[...] = xout
        xaug_ref[...] = jnp.concatenate(
            [xout, jnp.ones((_C, 1), jnp.float32)], axis=1)

        g = _dot_t(shi, shi) + _dot_t(shi, slo) + _dot_t(slo, shi)
        gnorm2 = jnp.sum(g * g)
        suma2 = jnp.sum(colsum)
        scal_ref[...] = jnp.concatenate(
            [jnp.reshape(suma2, (1, 1)), jnp.reshape(gnorm2, (1, 1))], axis=1)

    @pl.when(k >= _NBLK)
    def _pass2():
        j = k - _NBLK
        ab = a_ref[...].astype(jnp.bfloat16)
        y = (jnp.dot(ab, shi_ref[...], preferred_element_type=jnp.float32)
             + jnp.dot(ab, slo_ref[...], preferred_element_type=jnp.float32))
        yh, yl = _split(y)
        sh = shi_ref[pl.ds(j * _BLK, _BLK), :]
        sl = slo_ref[pl.ds(j * _BLK, _BLK), :]
        part = _dot_t(sh, yh) + _dot_t(sh, yl) + _dot_t(sl, yh)
        @pl.when(j == 0)
        def _():
            adj_ref[...] = part
        @pl.when(j > 0)
        def _():
            adj_ref[...] += part

    @pl.when(k == 2 * _NBLK - 1)
    def _final():
        adj_p = adj_ref[...]
        rows = jax.lax.broadcasted_iota(jnp.int32, (_C, _C), 0)
        cols = jax.lax.broadcasted_iota(jnp.int32, (_C, _C), 1)

        tr = jnp.sum(jnp.where(rows == cols, adj_p, 0.0))
        suma2 = scal_ref[0, 0]
        gnorm2 = scal_ref[0, 1]
        resid = jnp.maximum(suma2 - 2.0 * tr + gnorm2, 0.0)
        ll_ref[...] = jnp.reshape(jnp.sqrt(resid) / (_N * _N), (1, 1))

        row_max = jnp.max(adj_p, axis=1, keepdims=True)
        is_max = adj_p == row_max
        hard = is_max.astype(jnp.float32)
        arg_ref[...] = jnp.min(jnp.where(is_max, cols, _C), axis=1,
                               keepdims=True)

        xaug = xaug_ref[...]
        agg_aug = _dot_t(hard, xaug)
        deg3 = jnp.maximum(agg_aug[:, 2:3], 1.0)
        agg3 = agg_aug[:, 0:2] / deg3
        x3_ref[...] = _lin2(agg3, xaug[:, 0:2], w3l_ref[...], w3r_ref[...],
                            b3_ref[...])


def kernel(nodes, adjs, W1_l, W1_r, b1, W2_l, W2_r, b2, W3_l, W3_r, b3):
    naug = jnp.concatenate(
        [nodes, jnp.ones((_N, 1), jnp.float32)], axis=1)
    naug_hi = naug.astype(jnp.bfloat16)
    naug_lo = (naug - naug_hi.astype(jnp.float32)).astype(jnp.bfloat16)

    x3, arg, ll, ent, xout = pl.pallas_call(
        _body,
        grid=(2 * _NBLK,),
        in_specs=[
            pl.BlockSpec((_BLK, _N),
                         lambda k: (jax.lax.rem(k, _NBLK), 0)),
            pl.BlockSpec((_N, 3), lambda k: (0, 0)),
            pl.BlockSpec((_N, 3), lambda k: (0, 0)),
            pl.BlockSpec((2, 2), lambda k: (0, 0)),
            pl.BlockSpec((2, 2), lambda k: (0, 0)),
            pl.BlockSpec((1, 2), lambda k: (0, 0)),
            pl.BlockSpec((_C, 2), lambda k: (0, 0)),
            pl.BlockSpec((_C, 2), lambda k: (0, 0)),
            pl.BlockSpec((1, _C), lambda k: (0, 0)),
            pl.BlockSpec((1, 2), lambda k: (0, 0)),
            pl.BlockSpec((1, 2), lambda k: (0, 0)),
            pl.BlockSpec((1, 1), lambda k: (0, 0)),
        ],
        out_specs=[
            pl.BlockSpec((_C, 1), lambda k: (0, 0)),
            pl.BlockSpec((_C, 1), lambda k: (0, 0)),
            pl.BlockSpec((1, 1), lambda k: (0, 0)),
            pl.BlockSpec((1, 1), lambda k: (0, 0)),
            pl.BlockSpec((_C, 2), lambda k: (0, 0)),
        ],
        out_shape=[
            jax.ShapeDtypeStruct((_C, 1), jnp.float32),
            jax.ShapeDtypeStruct((_C, 1), jnp.int32),
            jax.ShapeDtypeStruct((1, 1), jnp.float32),
            jax.ShapeDtypeStruct((1, 1), jnp.float32),
            jax.ShapeDtypeStruct((_C, 2), jnp.float32),
        ],
        scratch_shapes=[
            pltpu.VMEM((_N, 3), jnp.float32),
            pltpu.VMEM((_N, _C), jnp.bfloat16),
            pltpu.VMEM((_N, _C), jnp.bfloat16),
            pltpu.VMEM((_C, _C), jnp.float32),
            pltpu.VMEM((1, 2), jnp.float32),
            pltpu.VMEM((_C, 3), jnp.float32),
        ],
        compiler_params=pltpu.CompilerParams(
            dimension_semantics=("arbitrary",)),
    )(adjs, naug_hi, naug_lo,
      W1_l, W1_r, b1.reshape(1, 2), W2_l, W2_r, b2.reshape(1, _C),
      W3_l, W3_r, b3.reshape(1, 1))

    x3_out = x3[:, 0]
    edge_index = jnp.stack(
        [jnp.arange(_C, dtype=jnp.int32), arg.reshape(_C)])
    return (x3_out, edge_index, ll.reshape(()), ent.reshape(()), xout)


# speedup vs baseline: 1.2351x; 1.0476x over previous
import jax
import jax.numpy as jnp
from jax.experimental import pallas as pl
from jax.experimental.pallas import tpu as pltpu

_N = 4096
_C = 128
_BLK = 1024
_NBLK = _N // _BLK


def _dot_t(a, b):
    return jax.lax.dot_general(
        a, b, (((0,), (0,)), ((), ())),
        preferred_element_type=jnp.float32)


def _split(x):
    hi = x.astype(jnp.bfloat16)
    lo = (x - hi.astype(jnp.float32)).astype(jnp.bfloat16)
    return hi, lo


def _lin2(a, b, Wl, Wr, bias):
    return (a[:, 0:1] * Wl[:, 0][None, :] + a[:, 1:2] * Wl[:, 1][None, :]
            + b[:, 0:1] * Wr[:, 0][None, :] + b[:, 1:2] * Wr[:, 1][None, :]
            + bias)


def _body(a_ref, nhi_ref, nlo_ref,
          w1l_ref, w1r_ref, b1_ref, w2l_ref, w2r_ref, b2_ref,
          w3l_ref, w3r_ref, b3_ref,
          x3_ref, arg_ref, ll_ref, ent_ref, xout_ref,
          atx_ref, shi_ref, slo_ref, adj_ref, scal_ref, xaug_ref):
    k = pl.program_id(0)

    @pl.when(k < _NBLK)
    def _pass1():
        base = k * _BLK
        ab = a_ref[...].astype(jnp.bfloat16)
        nh = nhi_ref[pl.ds(base, _BLK), :]
        nl = nlo_ref[pl.ds(base, _BLK), :]
        part = _dot_t(ab, nh) + _dot_t(ab, nl)
        @pl.when(k == 0)
        def _():
            atx_ref[...] = part
        @pl.when(k > 0)
        def _():
            atx_ref[...] += part

    @pl.when(k == _NBLK - 1)
    def _mid():
        atx = atx_ref[...]
        nodes = (nhi_ref[:, 0:2].astype(jnp.float32)
                 + nlo_ref[:, 0:2].astype(jnp.float32))
        colsum = atx[:, 2:3]
        deg = jnp.maximum(colsum, 1.0)
        agg = atx[:, 0:2] / deg

        x1 = _lin2(agg, nodes, w1l_ref[...], w1r_ref[...], b1_ref[...])
        logits = _lin2(agg, nodes, w2l_ref[...], w2r_ref[...], b2_ref[...])

        m = jnp.max(logits, axis=-1, keepdims=True)
        e = jnp.exp(logits - m)
        z = jnp.sum(e, axis=-1, keepdims=True)
        s = e / z
        shi, slo = _split(s)
        shi_ref[...] = shi
        slo_ref[...] = slo

        ent_rows = jnp.log(z) - jnp.sum(e * (logits - m), axis=-1,
                                        keepdims=True) / z
        ent_ref[...] = jnp.reshape(jnp.sum(ent_rows) / _N, (1, 1))

        x1h, x1l = _split(x1)
        xout = _dot_t(shi, x1h) + _dot_t(shi, x1l) + _dot_t(slo, x1h)
        xout_ref[...] = xout
        xaug_ref[...] = jnp.concatenate(
            [xout, jnp.ones((_C, 1), jnp.float32)], axis=1)

        g = _dot_t(shi, shi) + _dot_t(shi, slo) + _dot_t(slo, shi)
        gnorm2 = jnp.sum(g * g)
        suma2 = jnp.sum(colsum)
        scal_ref[...] = jnp.concatenate(
            [jnp.reshape(suma2, (1, 1)), jnp.reshape(gnorm2, (1, 1))], axis=1)

    @pl.when(k >= _NBLK)
    def _pass2():
        j = k - _NBLK
        ab = a_ref[...].astype(jnp.bfloat16)
        y = (jnp.dot(ab, shi_ref[...], preferred_element_type=jnp.float32)
             + jnp.dot(ab, slo_ref[...], preferred_element_type=jnp.float32))
        yh, yl = _split(y)
        sh = shi_ref[pl.ds(j * _BLK, _BLK), :]
        sl = slo_ref[pl.ds(j * _BLK, _BLK), :]
        part = _dot_t(sh, yh) + _dot_t(sh, yl) + _dot_t(sl, yh)
        @pl.when(j == 0)
        def _():
            adj_ref[...] = part
        @pl.when(j > 0)
        def _():
            adj_ref[...] += part

    @pl.when(k == 2 * _NBLK - 1)
    def _final():
        adj_p = adj_ref[...]
        rows = jax.lax.broadcasted_iota(jnp.int32, (_C, _C), 0)
        cols = jax.lax.broadcasted_iota(jnp.int32, (_C, _C), 1)

        tr = jnp.sum(jnp.where(rows == cols, adj_p, 0.0))
        suma2 = scal_ref[0, 0]
        gnorm2 = scal_ref[0, 1]
        resid = jnp.maximum(suma2 - 2.0 * tr + gnorm2, 0.0)
        ll_ref[...] = jnp.reshape(jnp.sqrt(resid) / (_N * _N), (1, 1))

        row_max = jnp.max(adj_p, axis=1, keepdims=True)
        is_max = adj_p == row_max
        hard = is_max.astype(jnp.float32)
        arg_ref[...] = jnp.min(jnp.where(is_max, cols, _C), axis=1,
                               keepdims=True)

        xaug = xaug_ref[...]
        agg_aug = _dot_t(hard, xaug)
        deg3 = jnp.maximum(agg_aug[:, 2:3], 1.0)
        agg3 = agg_aug[:, 0:2] / deg3
        x3_ref[...] = _lin2(agg3, xaug[:, 0:2], w3l_ref[...], w3r_ref[...],
                            b3_ref[...])


def kernel(nodes, adjs, W1_l, W1_r, b1, W2_l, W2_r, b2, W3_l, W3_r, b3):
    naug = jnp.concatenate(
        [nodes, jnp.ones((_N, 1), jnp.float32)], axis=1)
    naug_hi = naug.astype(jnp.bfloat16)
    naug_lo = (naug - naug_hi.astype(jnp.float32)).astype(jnp.bfloat16)

    x3, arg, ll, ent, xout = pl.pallas_call(
        _body,
        grid=(2 * _NBLK,),
        in_specs=[
            pl.BlockSpec((_BLK, _N),
                         lambda k: (jax.lax.rem(k, _NBLK), 0)),
            pl.BlockSpec((_N, 3), lambda k: (0, 0)),
            pl.BlockSpec((_N, 3), lambda k: (0, 0)),
            pl.BlockSpec((2, 2), lambda k: (0, 0)),
            pl.BlockSpec((2, 2), lambda k: (0, 0)),
            pl.BlockSpec((1, 2), lambda k: (0, 0)),
            pl.BlockSpec((_C, 2), lambda k: (0, 0)),
            pl.BlockSpec((_C, 2), lambda k: (0, 0)),
            pl.BlockSpec((1, _C), lambda k: (0, 0)),
            pl.BlockSpec((1, 2), lambda k: (0, 0)),
            pl.BlockSpec((1, 2), lambda k: (0, 0)),
            pl.BlockSpec((1, 1), lambda k: (0, 0)),
        ],
        out_specs=[
            pl.BlockSpec((_C, 1), lambda k: (0, 0)),
            pl.BlockSpec((_C, 1), lambda k: (0, 0)),
            pl.BlockSpec((1, 1), lambda k: (0, 0)),
            pl.BlockSpec((1, 1), lambda k: (0, 0)),
            pl.BlockSpec((_C, 2), lambda k: (0, 0)),
        ],
        out_shape=[
            jax.ShapeDtypeStruct((_C, 1), jnp.float32),
            jax.ShapeDtypeStruct((_C, 1), jnp.int32),
            jax.ShapeDtypeStruct((1, 1), jnp.float32),
            jax.ShapeDtypeStruct((1, 1), jnp.float32),
            jax.ShapeDtypeStruct((_C, 2), jnp.float32),
        ],
        scratch_shapes=[
            pltpu.VMEM((_N, 3), jnp.float32),
            pltpu.VMEM((_N, _C), jnp.bfloat16),
            pltpu.VMEM((_N, _C), jnp.bfloat16),
            pltpu.VMEM((_C, _C), jnp.float32),
            pltpu.VMEM((1, 2), jnp.float32),
            pltpu.VMEM((_C, 3), jnp.float32),
        ],
        compiler_params=pltpu.CompilerParams(
            dimension_semantics=("arbitrary",)),
    )(adjs, naug_hi, naug_lo,
      W1_l, W1_r, b1.reshape(1, 2), W2_l, W2_r, b2.reshape(1, _C),
      W3_l, W3_r, b3.reshape(1, 1))

    x3_out = x3[:, 0]
    edge_index = jnp.stack(
        [jnp.arange(_C, dtype=jnp.int32), arg.reshape(_C)])
    return (x3_out, edge_index, ll.reshape(()), ent.reshape(()), xout)
